# baseline (speedup 1.0000x reference)
import functools

import jax
import jax.numpy as jnp
from jax import lax
from jax.experimental import pallas as pl
from jax.experimental.pallas import tpu as pltpu
from jax.experimental.pallas import tpu_sc as plsc

N = 10000
D = 128
E = 320000
NP = 10240
NW = 32
ROWS_W = NP // NW
RCH = 80
NRCH = ROWS_W // RCH
EC = 10112
IB = 256


def _okey(bits):
    return bits ^ ((bits >> 31) & jnp.int32(0x7FFFFFFF))


def _rank_body(scol_ref, srow_ref, x_ref, rank_ref, y_ref, krow_ref):
    bi = pl.program_id(0)

    @pl.when(bi == 0)
    def _prep():
        kj = _okey(lax.bitcast_convert_type(srow_ref[...], jnp.int32))
        krow_ref[:, pl.ds(0, N)] = kj

    ki = _okey(lax.bitcast_convert_type(scol_ref[...], jnp.int32))
    krow = krow_ref[:, pl.ds(0, N)]
    jj = lax.broadcasted_iota(jnp.int32, (1, N), 1)
    adj = krow - (jj >= bi * IB).astype(jnp.int32)
    main = jnp.sum((adj >= ki).astype(jnp.int32), axis=1, keepdims=True)
    kd = krow_ref[:, pl.ds(bi * IB, IB)]
    jloc = lax.broadcasted_iota(jnp.int32, (1, IB), 1)
    iloc = lax.broadcasted_iota(jnp.int32, (IB, 1), 0)
    corr = jnp.sum(((kd == ki) & (jloc < iloc)).astype(jnp.int32),
                   axis=1, keepdims=True)
    rank_ref[...] = jnp.transpose(main + corr).reshape(IB)
    y_ref[...] = x_ref[...] * jnp.tanh(scol_ref[...])


EU = 4


def _sc_body(rank_hbm, y_hbm, edge_hbm, perm_hbm, nedge_hbm, gated_hbm,
             rank_v, ein_v, eout_v, ridx_v, iota_v, yrows_v,
             sem0, sem1, sem2, sem3):
    wid = lax.axis_index("s") * 2 + lax.axis_index("c")
    rbase = wid * ROWS_W
    ebase = jnp.minimum(wid * EC, E - EC)
    is_last = wid == NW - 1

    h_rank = pltpu.async_copy(rank_hbm, rank_v, sem0)
    h_ein = pltpu.async_copy(edge_hbm.at[:, pl.ds(ebase, EC)], ein_v, sem1)
    hs = []
    for c in range(NRCH):
        sel = jnp.where(is_last, 0, c) if c else 0
        base = rbase + sel * RCH
        hs.append(pltpu.async_copy(y_hbm.at[pl.ds(base, RCH)],
                                   yrows_v.at[pl.ds(c * RCH, RCH)], sem2))
        hs.append(pltpu.async_copy(rank_hbm.at[pl.ds(base, RCH)],
                                   ridx_v.at[c], sem2))
        for k in range(RCH // 16):
            iota_v[c, pl.ds(k * 16, 16)] = (
                lax.iota(jnp.int32, 16) + (base + k * 16))

    for h in hs:
        h.wait()
    hsc = []
    for c in range(NRCH):
        hsc.append(pltpu.async_copy(iota_v.at[c],
                                    perm_hbm.at[ridx_v.at[c]], sem3))
        hsc.append(pltpu.async_copy(yrows_v.at[pl.ds(c * RCH, RCH)],
                                    gated_hbm.at[ridx_v.at[c]], sem3))

    h_rank.wait()
    h_ein.wait()

    def edge_step(i, carry):
        b = i * (16 * EU)
        for u in range(EU):
            idx0 = ein_v[0, pl.ds(b + u * 16, 16)]
            eout_v[0, pl.ds(b + u * 16, 16)] = plsc.load_gather(rank_v, [idx0])
            idx1 = ein_v[1, pl.ds(b + u * 16, 16)]
            eout_v[1, pl.ds(b + u * 16, 16)] = plsc.load_gather(rank_v, [idx1])
        return carry

    lax.fori_loop(0, EC // (16 * EU), edge_step, 0)
    h_eout = pltpu.async_copy(eout_v, nedge_hbm.at[:, pl.ds(ebase, EC)], sem1)

    for h in hsc:
        h.wait()
    h_eout.wait()


@functools.cache
def _sc_kernel_build():
    return functools.partial(
        pl.kernel,
        mesh=plsc.VectorSubcoreMesh(core_axis_name="c", subcore_axis_name="s"),
        compiler_params=pltpu.CompilerParams(needs_layout_passes=False),
        out_type=[
            jax.ShapeDtypeStruct((N,), jnp.int32),
            jax.ShapeDtypeStruct((2, E), jnp.int32),
            jax.ShapeDtypeStruct((N, D), jnp.float32),
        ],
        scratch_types=[
            pltpu.VMEM((NP,), jnp.int32),
            pltpu.VMEM((2, EC), jnp.int32),
            pltpu.VMEM((2, EC), jnp.int32),
            pltpu.VMEM((NRCH, RCH), jnp.int32),
            pltpu.VMEM((NRCH, RCH), jnp.int32),
            pltpu.VMEM((ROWS_W, D), jnp.float32),
            pltpu.SemaphoreType.DMA,
            pltpu.SemaphoreType.DMA,
            pltpu.SemaphoreType.DMA,
            pltpu.SemaphoreType.DMA,
        ],
    )(_sc_body)


def kernel(x, edge_index, epoch, W, b):
    del epoch
    x = x.astype(jnp.float32)
    edt = edge_index.dtype

    scores = (x @ W.T + b).reshape(-1)
    scores = scores / jnp.linalg.norm(W)
    nxb = (N - 1) // IB + 1
    rank, y = pl.pallas_call(
        _rank_body,
        grid=(NP // IB,),
        in_specs=[
            pl.BlockSpec((IB, 1), lambda i: (jnp.minimum(i, nxb - 1), 0)),
            pl.BlockSpec((1, N), lambda i: (0, 0)),
            pl.BlockSpec((IB, D), lambda i: (jnp.minimum(i, nxb - 1), 0)),
        ],
        out_specs=[
            pl.BlockSpec((IB,), lambda i: (i,)),
            pl.BlockSpec((IB, D), lambda i: (i, 0)),
        ],
        out_shape=[
            jax.ShapeDtypeStruct((NP,), jnp.int32),
            jax.ShapeDtypeStruct((NP, D), jnp.float32),
        ],
        scratch_shapes=[pltpu.VMEM((1, NP), jnp.int32)],
    )(scores.reshape(N, 1), scores.reshape(1, N), x)

    perm, new_edge_index, gated = _sc_kernel_build()(
        rank, y, edge_index.astype(jnp.int32))
    return gated, new_edge_index.astype(edt), scores, perm


# speedup vs baseline: 1.0209x; 1.0209x over previous
import functools

import jax
import jax.numpy as jnp
from jax import lax
from jax.experimental import pallas as pl
from jax.experimental.pallas import tpu as pltpu
from jax.experimental.pallas import tpu_sc as plsc

N = 10000
D = 128
E = 320000
NP = 10240
NW = 32
ROWS_W = NP // NW
RCH = 80
NRCH = ROWS_W // RCH
EC = 10112
IB = 256


def _okey(bits):
    return bits ^ ((bits >> 31) & jnp.int32(0x7FFFFFFF))


def _rank_body(scol_ref, srow_ref, x_ref, rank_ref, y_ref, krow_ref):
    bi = pl.program_id(0)

    @pl.when(bi == 0)
    def _prep():
        jj0 = lax.broadcasted_iota(jnp.int32, (1, NP), 1)
        kj = _okey(lax.bitcast_convert_type(srow_ref[...], jnp.int32))
        krow_ref[...] = kj
        del jj0

    ki = _okey(lax.bitcast_convert_type(scol_ref[...], jnp.int32))
    krow = krow_ref[...]
    jj = lax.broadcasted_iota(jnp.int32, (1, NP), 1)
    adj = krow - (jj >= bi * IB).astype(jnp.int32)
    main = jnp.sum((adj >= ki).astype(jnp.int32), axis=1, keepdims=True)
    kd = krow_ref[:, pl.ds(bi * IB, IB)]
    jloc = lax.broadcasted_iota(jnp.int32, (1, IB), 1)
    iloc = lax.broadcasted_iota(jnp.int32, (IB, 1), 0)
    corr = jnp.sum(((kd == ki) & (jloc < iloc)).astype(jnp.int32),
                   axis=1, keepdims=True)
    rank_ref[...] = jnp.transpose(main + corr).reshape(IB)
    y_ref[...] = x_ref[...] * jnp.tanh(scol_ref[...])


EU = 4


def _sc_body(rank_hbm, y_hbm, edge_hbm, perm_hbm, nedge_hbm, gated_hbm,
             rank_v, ein_v, eout_v, ridx_v, iota_v, yrows_v,
             sem0, sem1, sem2, sem3):
    wid = lax.axis_index("s") * 2 + lax.axis_index("c")
    rbase = wid * ROWS_W
    ebase = jnp.minimum(wid * EC, E - EC)
    is_last = wid == NW - 1

    h_rank = pltpu.async_copy(rank_hbm, rank_v, sem0)
    h_ein = pltpu.async_copy(edge_hbm.at[:, pl.ds(ebase, EC)], ein_v, sem1)
    hs = []
    for c in range(NRCH):
        sel = jnp.where(is_last, 0, c) if c else 0
        base = rbase + sel * RCH
        hs.append(pltpu.async_copy(y_hbm.at[pl.ds(base, RCH)],
                                   yrows_v.at[pl.ds(c * RCH, RCH)], sem2))
        hs.append(pltpu.async_copy(rank_hbm.at[pl.ds(base, RCH)],
                                   ridx_v.at[c], sem2))
        for k in range(RCH // 16):
            iota_v[c, pl.ds(k * 16, 16)] = (
                lax.iota(jnp.int32, 16) + (base + k * 16))

    for h in hs:
        h.wait()
    hsc = []
    for c in range(NRCH):
        hsc.append(pltpu.async_copy(iota_v.at[c],
                                    perm_hbm.at[ridx_v.at[c]], sem3))
        hsc.append(pltpu.async_copy(yrows_v.at[pl.ds(c * RCH, RCH)],
                                    gated_hbm.at[ridx_v.at[c]], sem3))

    h_rank.wait()
    h_ein.wait()

    def edge_step(i, carry):
        b = i * (16 * EU)
        for u in range(EU):
            idx0 = ein_v[0, pl.ds(b + u * 16, 16)]
            eout_v[0, pl.ds(b + u * 16, 16)] = plsc.load_gather(rank_v, [idx0])
            idx1 = ein_v[1, pl.ds(b + u * 16, 16)]
            eout_v[1, pl.ds(b + u * 16, 16)] = plsc.load_gather(rank_v, [idx1])
        return carry

    lax.fori_loop(0, EC // (16 * EU), edge_step, 0)
    h_eout = pltpu.async_copy(eout_v, nedge_hbm.at[:, pl.ds(ebase, EC)], sem1)

    for h in hsc:
        h.wait()
    h_eout.wait()


@functools.cache
def _sc_kernel_build():
    return functools.partial(
        pl.kernel,
        mesh=plsc.VectorSubcoreMesh(core_axis_name="c", subcore_axis_name="s"),
        compiler_params=pltpu.CompilerParams(needs_layout_passes=False),
        out_type=[
            jax.ShapeDtypeStruct((N,), jnp.int32),
            jax.ShapeDtypeStruct((2, E), jnp.int32),
            jax.ShapeDtypeStruct((N, D), jnp.float32),
        ],
        scratch_types=[
            pltpu.VMEM((NP,), jnp.int32),
            pltpu.VMEM((2, EC), jnp.int32),
            pltpu.VMEM((2, EC), jnp.int32),
            pltpu.VMEM((NRCH, RCH), jnp.int32),
            pltpu.VMEM((NRCH, RCH), jnp.int32),
            pltpu.VMEM((ROWS_W, D), jnp.float32),
            pltpu.SemaphoreType.DMA,
            pltpu.SemaphoreType.DMA,
            pltpu.SemaphoreType.DMA,
            pltpu.SemaphoreType.DMA,
        ],
    )(_sc_body)


def kernel(x, edge_index, epoch, W, b):
    del epoch
    x = x.astype(jnp.float32)
    edt = edge_index.dtype

    scores = (x @ W.T + b).reshape(-1)
    scores = scores / jnp.linalg.norm(W)
    s_pad = jnp.full((NP,), -jnp.inf, jnp.float32).at[:N].set(scores)

    nxb = (N - 1) // IB + 1
    rank, y = pl.pallas_call(
        _rank_body,
        grid=(NP // IB,),
        in_specs=[
            pl.BlockSpec((IB, 1), lambda i: (i, 0)),
            pl.BlockSpec((1, NP), lambda i: (0, 0)),
            pl.BlockSpec((IB, D), lambda i: (jnp.minimum(i, nxb - 1), 0)),
        ],
        out_specs=[
            pl.BlockSpec((IB,), lambda i: (i,)),
            pl.BlockSpec((IB, D), lambda i: (i, 0)),
        ],
        out_shape=[
            jax.ShapeDtypeStruct((NP,), jnp.int32),
            jax.ShapeDtypeStruct((NP, D), jnp.float32),
        ],
        scratch_shapes=[pltpu.VMEM((1, NP), jnp.int32)],
    )(s_pad.reshape(NP, 1), s_pad.reshape(1, NP), x)

    perm, new_edge_index, gated = _sc_kernel_build()(
        rank, y, edge_index.astype(jnp.int32))
    return gated, new_edge_index.astype(edt), scores, perm
